# Initial kernel scaffold; baseline (speedup 1.0000x reference)
import functools

import jax
import jax.numpy as jnp
from jax import lax
from jax.experimental import pallas as pl
from jax.experimental.pallas import tpu as pltpu
from jax.experimental.pallas import tpu_sc as plsc

N_NODES = 10000
N_EDGES = 320000
D = 128
NC = 2
NS = 16
NW = NC * NS
CHUNK = 128
E_PER_TILE = -(-N_EDGES // NW // CHUNK) * CHUNK
NCH = E_PER_TILE // CHUNK
E_PAD = NW * E_PER_TILE
ROWS_PER_TILE = N_NODES // NS


def _sc_aggregate(x_pad, src, dst):
    mesh = plsc.VectorSubcoreMesh(core_axis_name="c", subcore_axis_name="s")

    @functools.partial(
        pl.kernel,
        mesh=mesh,
        out_type=jax.ShapeDtypeStruct((NC, N_NODES, D), jnp.float32),
        scratch_types=[
            pltpu.VMEM((NCH, CHUNK), jnp.int32),
            pltpu.VMEM((NCH, CHUNK), jnp.int32),
            pltpu.VMEM((CHUNK, D), jnp.float32),
            pltpu.VMEM_SHARED((N_NODES, D), jnp.float32),
            pltpu.SemaphoreType.DMA,
        ],
    )
    def k(x_hbm, src_hbm, dst_hbm, out_hbm, src_v, dst_v, rows_v, acc, sem):
        c = lax.axis_index("c")
        s = lax.axis_index("s")
        wid = s * NC + c

        pltpu.sync_copy(src_hbm.at[wid], src_v)
        pltpu.sync_copy(dst_hbm.at[wid], dst_v)

        @pl.when(s == 0)
        def _():
            pltpu.sync_copy(x_hbm.at[pl.ds(0, N_NODES)], acc)

        plsc.subcore_barrier()

        def body(j, carry):
            pltpu.async_copy(x_hbm.at[src_v.at[j]], rows_v, sem).wait()
            pltpu.sync_copy(rows_v, acc.at[dst_v.at[j]], add=True)
            return carry

        lax.fori_loop(0, NCH, body, 0)

        plsc.subcore_barrier()

        pltpu.sync_copy(
            acc.at[pl.ds(s * ROWS_PER_TILE, ROWS_PER_TILE)],
            out_hbm.at[c, pl.ds(s * ROWS_PER_TILE, ROWS_PER_TILE)],
        )

    return k(x_pad, src, dst)


def _mlp_body(p_ref, x_ref, w1_ref, b1_ref, g_ref, be_ref, w2_ref, b2_ref,
              o_ref):
    h = p_ref[0] + p_ref[1] - x_ref[...]
    h1 = jnp.dot(h, w1_ref[...], preferred_element_type=jnp.float32)
    h1 = h1 + b1_ref[...]
    mean = jnp.mean(h1, axis=0, keepdims=True)
    var = jnp.mean((h1 - mean) * (h1 - mean), axis=0, keepdims=True)
    hn = g_ref[...] * (h1 - mean) * lax.rsqrt(var + 1e-5) + be_ref[...]
    hn = jnp.maximum(hn, 0.0)
    o_ref[...] = (
        jnp.dot(hn, w2_ref[...], preferred_element_type=jnp.float32)
        + b2_ref[...]
    )


def _mlp(p, x, W1, b1, gamma, beta, W2, b2):
    return pl.pallas_call(
        _mlp_body,
        out_shape=jax.ShapeDtypeStruct((N_NODES, D), jnp.float32),
    )(p, x, W1, b1.reshape(1, D), gamma.reshape(1, D), beta.reshape(1, D),
      W2, b2.reshape(1, D))


def kernel(x, edge_index, edge_attr, W1, b1, gamma, beta, W2, b2):
    src = edge_index[0].astype(jnp.int32)
    dst = edge_index[1].astype(jnp.int32)
    pad = E_PAD - N_EDGES
    src_p = jnp.concatenate(
        [src, jnp.full((pad,), N_NODES, jnp.int32)]).reshape(NW, NCH, CHUNK)
    dst_p = jnp.concatenate(
        [dst, jnp.zeros((pad,), jnp.int32)]).reshape(NW, NCH, CHUNK)
    x_pad = jnp.concatenate([x, jnp.zeros((1, D), jnp.float32)], axis=0)
    partials = _sc_aggregate(x_pad, src_p, dst_p)
    return _mlp(partials, x, W1, b1, gamma, beta, W2, b2)


# speedup vs baseline: 4.3845x; 4.3845x over previous
import functools

import jax
import jax.numpy as jnp
from jax import lax
from jax.experimental import pallas as pl
from jax.experimental.pallas import tpu as pltpu
from jax.experimental.pallas import tpu_sc as plsc

N_NODES = 10000
N_EDGES = 320000
D = 128
NC = 2
NS = 16
NW = NC * NS
CHUNK = 128
E_PER_TILE = -(-N_EDGES // NW // CHUNK) * CHUNK
NCH = E_PER_TILE // CHUNK
E_PAD = NW * E_PER_TILE
ROWS_PER_TILE = 632
N_PAD = NS * ROWS_PER_TILE


def _sc_aggregate(x_pad, src, dst):
    mesh = plsc.VectorSubcoreMesh(core_axis_name="c", subcore_axis_name="s")

    @functools.partial(
        pl.kernel,
        mesh=mesh,
        out_type=jax.ShapeDtypeStruct((NC, N_PAD, D), jnp.float32),
        scratch_types=[
            pltpu.VMEM((NCH, CHUNK), jnp.int32),
            pltpu.VMEM((NCH, CHUNK), jnp.int32),
            pltpu.VMEM((CHUNK, D), jnp.float32),
            pltpu.VMEM_SHARED((N_PAD, D), jnp.float32),
            pltpu.SemaphoreType.DMA,
        ],
    )
    def k(x_hbm, src_hbm, dst_hbm, out_hbm, src_v, dst_v, rows_v, acc, sem):
        c = lax.axis_index("c")
        s = lax.axis_index("s")
        wid = s * NC + c

        pltpu.sync_copy(src_hbm.at[wid], src_v)
        pltpu.sync_copy(dst_hbm.at[wid], dst_v)

        @pl.when(s == 0)
        def _():
            pltpu.sync_copy(x_hbm, acc)

        plsc.subcore_barrier()

        def body(j, carry):
            pltpu.async_copy(x_hbm.at[src_v.at[j]], rows_v, sem).wait()
            pltpu.sync_copy(rows_v, acc.at[dst_v.at[j]], add=True)
            return carry

        lax.fori_loop(0, NCH, body, 0)

        plsc.subcore_barrier()

        pltpu.sync_copy(
            acc.at[pl.ds(s * ROWS_PER_TILE, ROWS_PER_TILE)],
            out_hbm.at[c, pl.ds(s * ROWS_PER_TILE, ROWS_PER_TILE)],
        )

    return k(x_pad, src, dst)


def _mlp_body(p_ref, x_ref, w1_ref, b1_ref, g_ref, be_ref, w2_ref, b2_ref,
              o_ref):
    h = p_ref[0][:N_NODES] + p_ref[1][:N_NODES] - x_ref[...]
    h1 = jnp.dot(h, w1_ref[...], preferred_element_type=jnp.float32)
    h1 = h1 + b1_ref[...]
    mean = jnp.mean(h1, axis=0, keepdims=True)
    var = jnp.mean((h1 - mean) * (h1 - mean), axis=0, keepdims=True)
    hn = g_ref[...] * (h1 - mean) * lax.rsqrt(var + 1e-5) + be_ref[...]
    hn = jnp.maximum(hn, 0.0)
    o_ref[...] = (
        jnp.dot(hn, w2_ref[...], preferred_element_type=jnp.float32)
        + b2_ref[...]
    )


def _mlp(p, x, W1, b1, gamma, beta, W2, b2):
    return pl.pallas_call(
        _mlp_body,
        out_shape=jax.ShapeDtypeStruct((N_NODES, D), jnp.float32),
    )(p, x, W1, b1.reshape(1, D), gamma.reshape(1, D), beta.reshape(1, D),
      W2, b2.reshape(1, D))


def kernel(x, edge_index, edge_attr, W1, b1, gamma, beta, W2, b2):
    src = edge_index[0].astype(jnp.int32)
    dst = edge_index[1].astype(jnp.int32)
    pad = E_PAD - N_EDGES
    src_p = jnp.concatenate(
        [src, jnp.full((pad,), N_NODES, jnp.int32)]).reshape(NW, NCH, CHUNK)
    dst_p = jnp.concatenate(
        [dst, jnp.zeros((pad,), jnp.int32)]).reshape(NW, NCH, CHUNK)
    x_pad = jnp.concatenate(
        [x, jnp.zeros((N_PAD - N_NODES, D), jnp.float32)], axis=0)
    partials = _sc_aggregate(x_pad, src_p, dst_p)
    return _mlp(partials, x, W1, b1, gamma, beta, W2, b2)
